# baseline (speedup 1.0000x reference)
import jax
import jax.numpy as jnp
from jax import lax
from jax.experimental import pallas as pl
from jax.experimental.pallas import tpu as pltpu
from jax.experimental.pallas import tpu_sc as plsc

VOCAB = 1000000
DIM = 64
DIM_PAD = 128
B = 16384
L = 20

NC = 2
NS = 16
LANES = 16
NW = NC * NS

BPW = B // NW
CB = 32
NCHUNK = BPW // CB
OUT_PER_CHUNK = CB * L
CTX_IROWS = OUT_PER_CHUNK // 128

TCOLS = 512
TGRID = (VOCAB + TCOLS - 1) // TCOLS


def _tp_body(x_ref, o_ref):
    t = jnp.transpose(x_ref[...], (1, 0))
    o_ref[...] = jnp.concatenate(
        [t, jnp.zeros((TCOLS, DIM_PAD - DIM), jnp.float32)], axis=1)


def _transpose_pad(table_t):
    return pl.pallas_call(
        _tp_body,
        grid=(TGRID,),
        in_specs=[pl.BlockSpec((DIM, TCOLS), lambda i: (0, i))],
        out_specs=pl.BlockSpec((TCOLS, DIM_PAD), lambda i: (i, 0)),
        out_shape=jax.ShapeDtypeStruct((VOCAB, DIM_PAD), jnp.float32),
    )(table_t)


def _sc_body(wids_r, cids_r, wtab_r, ctab_r, out_r,
             idxw_v, idxc_v, wrows_v, crows_v, outv, semw, semc):
    c = lax.axis_index("c")
    s = lax.axis_index("s")
    wid = s * NC + c
    lane15 = lax.iota(jnp.int32, LANES) == (LANES - 1)

    def chunk_body(k, carry):
        g = wid * NCHUNK + k
        pltpu.sync_copy(wids_r.at[pl.ds(g * CB, CB)], idxw_v)
        for j in range(CTX_IROWS):
            pltpu.sync_copy(
                cids_r.at[pl.ds(g * OUT_PER_CHUNK + j * 128, 128)],
                idxc_v.at[j])
        cw = pltpu.async_copy(wtab_r.at[idxw_v], wrows_v, semw)
        ccs = []
        for j in range(CTX_IROWS):
            ccs.append(pltpu.async_copy(
                ctab_r.at[idxc_v.at[j]],
                crows_v.at[pl.ds(j * 128, 128)], semc))
        cw.wait()
        for cc in ccs:
            cc.wait()

        def word_body(b, carry2):
            wv = [wrows_v[b, pl.ds(h * LANES, LANES)] for h in range(4)]
            for l in range(L):
                o = b * L + l
                p01 = (wv[0] * crows_v[o, pl.ds(0, LANES)]
                       + wv[1] * crows_v[o, pl.ds(LANES, LANES)])
                p23 = (wv[2] * crows_v[o, pl.ds(2 * LANES, LANES)]
                       + wv[3] * crows_v[o, pl.ds(3 * LANES, LANES)])
                cum = plsc.cumsum(p01 + p23)
                plsc.store_scatter(
                    outv, [jnp.broadcast_to(o, (LANES,))], cum, mask=lane15)
            return carry2

        lax.fori_loop(0, CB, word_body, 0)
        pltpu.sync_copy(outv, out_r.at[pl.ds(g * OUT_PER_CHUNK, OUT_PER_CHUNK)])
        return carry

    lax.fori_loop(0, NCHUNK, chunk_body, 0)


@jax.jit
def kernel(word_ids, context_ids, word_embed, context_embed):
    wids = word_ids.astype(jnp.int32).reshape(B)
    cids = context_ids.astype(jnp.int32).reshape(B * L)
    wtab = _transpose_pad(word_embed.T)
    ctab = _transpose_pad(context_embed.T)

    mesh = plsc.VectorSubcoreMesh(core_axis_name="c", subcore_axis_name="s")
    out_flat = pl.kernel(
        _sc_body,
        out_type=jax.ShapeDtypeStruct((B * L,), jnp.float32),
        mesh=mesh,
        scratch_types=[
            pltpu.VMEM((CB,), jnp.int32),
            pltpu.VMEM((CTX_IROWS, 128), jnp.int32),
            pltpu.VMEM((CB, DIM_PAD), jnp.float32),
            pltpu.VMEM((OUT_PER_CHUNK, DIM_PAD), jnp.float32),
            pltpu.VMEM((OUT_PER_CHUNK,), jnp.float32),
            pltpu.SemaphoreType.DMA,
            pltpu.SemaphoreType.DMA,
        ],
        compiler_params=pltpu.CompilerParams(
            needs_layout_passes=False, use_tc_tiling_on_sc=True),
    )(wids, cids, wtab, ctab)
    return out_flat.reshape(B, L)


# speedup vs baseline: 2.2008x; 2.2008x over previous
import jax
import jax.numpy as jnp
from jax import lax
from jax.experimental import pallas as pl
from jax.experimental.pallas import tpu as pltpu
from jax.experimental.pallas import tpu_sc as plsc

VOCAB = 1000000
DIM = 64
DIM_PAD = 128
B = 16384
L = 20

NC = 2
NS = 16
LANES = 16
NW = NC * NS

BPW = B // NW
CB = 32
NCHUNK = BPW // CB
OUT_PER_CHUNK = CB * L
CTX_IROWS = OUT_PER_CHUNK // 128

TCOLS = 2048
TGRID = (VOCAB + TCOLS - 1) // TCOLS


def _tp_body(x_ref, o_ref):
    t = jnp.transpose(x_ref[...], (1, 0))
    o_ref[...] = jnp.concatenate(
        [t, jnp.zeros((TCOLS, DIM_PAD - DIM), jnp.float32)], axis=1)


def _transpose_pad(table_t):
    return pl.pallas_call(
        _tp_body,
        grid=(TGRID,),
        in_specs=[pl.BlockSpec((DIM, TCOLS), lambda i: (0, i))],
        out_specs=pl.BlockSpec((TCOLS, DIM_PAD), lambda i: (i, 0)),
        out_shape=jax.ShapeDtypeStruct((VOCAB, DIM_PAD), jnp.float32),
    )(table_t)


def _sc_body(wids_r, cids_r, wtab_r, ctab_r, out_r,
             idxw_v, idxc_v, wrows_v, crows_v, outv, semw, semc):
    c = lax.axis_index("c")
    s = lax.axis_index("s")
    wid = s * NC + c
    lane15 = lax.iota(jnp.int32, LANES) == (LANES - 1)

    def chunk_body(k, carry):
        g = wid * NCHUNK + k
        pltpu.sync_copy(wids_r.at[pl.ds(g * CB, CB)], idxw_v)
        for j in range(CTX_IROWS):
            pltpu.sync_copy(
                cids_r.at[pl.ds(g * OUT_PER_CHUNK + j * 128, 128)],
                idxc_v.at[j])
        cw = pltpu.async_copy(wtab_r.at[idxw_v], wrows_v, semw)
        ccs = []
        for j in range(CTX_IROWS):
            ccs.append(pltpu.async_copy(
                ctab_r.at[idxc_v.at[j]],
                crows_v.at[pl.ds(j * 128, 128)], semc))
        cw.wait()
        for cc in ccs:
            cc.wait()

        def word_body(b, carry2):
            wv = [wrows_v[b, pl.ds(h * LANES, LANES)] for h in range(4)]
            for l in range(L):
                o = b * L + l
                p01 = (wv[0] * crows_v[o, pl.ds(0, LANES)]
                       + wv[1] * crows_v[o, pl.ds(LANES, LANES)])
                p23 = (wv[2] * crows_v[o, pl.ds(2 * LANES, LANES)]
                       + wv[3] * crows_v[o, pl.ds(3 * LANES, LANES)])
                cum = plsc.cumsum(p01 + p23)
                plsc.store_scatter(
                    outv, [jnp.broadcast_to(o, (LANES,))], cum, mask=lane15)
            return carry2

        lax.fori_loop(0, CB, word_body, 0)
        pltpu.sync_copy(outv, out_r.at[pl.ds(g * OUT_PER_CHUNK, OUT_PER_CHUNK)])
        return carry

    lax.fori_loop(0, NCHUNK, chunk_body, 0)


@jax.jit
def kernel(word_ids, context_ids, word_embed, context_embed):
    wids = word_ids.astype(jnp.int32).reshape(B)
    cids = context_ids.astype(jnp.int32).reshape(B * L)
    wtab = _transpose_pad(word_embed.T)
    ctab = _transpose_pad(context_embed.T)

    mesh = plsc.VectorSubcoreMesh(core_axis_name="c", subcore_axis_name="s")
    out_flat = pl.kernel(
        _sc_body,
        out_type=jax.ShapeDtypeStruct((B * L,), jnp.float32),
        mesh=mesh,
        scratch_types=[
            pltpu.VMEM((CB,), jnp.int32),
            pltpu.VMEM((CTX_IROWS, 128), jnp.int32),
            pltpu.VMEM((CB, DIM_PAD), jnp.float32),
            pltpu.VMEM((OUT_PER_CHUNK, DIM_PAD), jnp.float32),
            pltpu.VMEM((OUT_PER_CHUNK,), jnp.float32),
            pltpu.SemaphoreType.DMA,
            pltpu.SemaphoreType.DMA,
        ],
        compiler_params=pltpu.CompilerParams(
            needs_layout_passes=False, use_tc_tiling_on_sc=True),
    )(wids, cids, wtab, ctab)
    return out_flat.reshape(B, L)


# speedup vs baseline: 3.2296x; 1.4675x over previous
import jax
import jax.numpy as jnp
from jax import lax
from jax.experimental import pallas as pl
from jax.experimental.pallas import tpu as pltpu
from jax.experimental.pallas import tpu_sc as plsc

VOCAB = 1000000
DIM = 64
DIM_PAD = 128
B = 16384
L = 20

NC = 2
NS = 16
LANES = 16
NW = NC * NS

BPW = B // NW
CB = 32
NCHUNK = BPW // CB
OUT_PER_CHUNK = CB * L
CTX_IROWS = OUT_PER_CHUNK // 128

TCOLS = 8192
TGRID = (VOCAB + TCOLS - 1) // TCOLS


def _tp_body(x_ref, o_ref):
    t = jnp.transpose(x_ref[...], (1, 0))
    o_ref[...] = jnp.concatenate(
        [t, jnp.zeros((TCOLS, DIM_PAD - DIM), jnp.float32)], axis=1)


def _transpose_pad(table_t):
    return pl.pallas_call(
        _tp_body,
        grid=(TGRID,),
        in_specs=[pl.BlockSpec((DIM, TCOLS), lambda i: (0, i))],
        out_specs=pl.BlockSpec((TCOLS, DIM_PAD), lambda i: (i, 0)),
        out_shape=jax.ShapeDtypeStruct((VOCAB, DIM_PAD), jnp.float32),
    )(table_t)


def _sc_body(wids_r, cids_r, wtab_r, ctab_r, out_r,
             idxw_v, idxc_v, wrows_v, crows_v, outv, semw, semc):
    c = lax.axis_index("c")
    s = lax.axis_index("s")
    wid = s * NC + c
    lane15 = lax.iota(jnp.int32, LANES) == (LANES - 1)

    def chunk_body(k, carry):
        g = wid * NCHUNK + k
        pltpu.sync_copy(wids_r.at[pl.ds(g * CB, CB)], idxw_v)
        for j in range(CTX_IROWS):
            pltpu.sync_copy(
                cids_r.at[pl.ds(g * OUT_PER_CHUNK + j * 128, 128)],
                idxc_v.at[j])
        cw = pltpu.async_copy(wtab_r.at[idxw_v], wrows_v, semw)
        ccs = []
        for j in range(CTX_IROWS):
            ccs.append(pltpu.async_copy(
                ctab_r.at[idxc_v.at[j]],
                crows_v.at[pl.ds(j * 128, 128)], semc))
        cw.wait()
        for cc in ccs:
            cc.wait()

        def word_body(b, carry2):
            wv = [wrows_v[b, pl.ds(h * LANES, LANES)] for h in range(4)]
            for l in range(L):
                o = b * L + l
                p01 = (wv[0] * crows_v[o, pl.ds(0, LANES)]
                       + wv[1] * crows_v[o, pl.ds(LANES, LANES)])
                p23 = (wv[2] * crows_v[o, pl.ds(2 * LANES, LANES)]
                       + wv[3] * crows_v[o, pl.ds(3 * LANES, LANES)])
                cum = plsc.cumsum(p01 + p23)
                plsc.store_scatter(
                    outv, [jnp.broadcast_to(o, (LANES,))], cum, mask=lane15)
            return carry2

        lax.fori_loop(0, CB, word_body, 0)
        pltpu.sync_copy(outv, out_r.at[pl.ds(g * OUT_PER_CHUNK, OUT_PER_CHUNK)])
        return carry

    lax.fori_loop(0, NCHUNK, chunk_body, 0)


@jax.jit
def kernel(word_ids, context_ids, word_embed, context_embed):
    wids = word_ids.astype(jnp.int32).reshape(B)
    cids = context_ids.astype(jnp.int32).reshape(B * L)
    wtab = _transpose_pad(word_embed.T)
    ctab = _transpose_pad(context_embed.T)

    mesh = plsc.VectorSubcoreMesh(core_axis_name="c", subcore_axis_name="s")
    out_flat = pl.kernel(
        _sc_body,
        out_type=jax.ShapeDtypeStruct((B * L,), jnp.float32),
        mesh=mesh,
        scratch_types=[
            pltpu.VMEM((CB,), jnp.int32),
            pltpu.VMEM((CTX_IROWS, 128), jnp.int32),
            pltpu.VMEM((CB, DIM_PAD), jnp.float32),
            pltpu.VMEM((OUT_PER_CHUNK, DIM_PAD), jnp.float32),
            pltpu.VMEM((OUT_PER_CHUNK,), jnp.float32),
            pltpu.SemaphoreType.DMA,
            pltpu.SemaphoreType.DMA,
        ],
        compiler_params=pltpu.CompilerParams(
            needs_layout_passes=False, use_tc_tiling_on_sc=True),
    )(wids, cids, wtab, ctab)
    return out_flat.reshape(B, L)


# speedup vs baseline: 3.6987x; 1.1452x over previous
import jax
import jax.numpy as jnp
from jax import lax
from jax.experimental import pallas as pl
from jax.experimental.pallas import tpu as pltpu
from jax.experimental.pallas import tpu_sc as plsc

VOCAB = 1000000
DIM = 64
DIM_PAD = 128
B = 16384
L = 20

NC = 2
NS = 16
LANES = 16
NW = NC * NS

BPW = B // NW
CB = 16
NCHUNK = BPW // CB
OUT_PER_CHUNK = CB * L
CTX_PER_LAUNCH = 64
CTX_LAUNCH = OUT_PER_CHUNK // CTX_PER_LAUNCH

TCOLS = 8192
TGRID = (VOCAB + TCOLS - 1) // TCOLS


def _tp_body(x_ref, o_ref):
    t = jnp.transpose(x_ref[...], (1, 0))
    o_ref[...] = jnp.concatenate(
        [t, jnp.zeros((TCOLS, DIM_PAD - DIM), jnp.float32)], axis=1)


def _transpose_pad(table_t):
    return pl.pallas_call(
        _tp_body,
        grid=(TGRID,),
        in_specs=[pl.BlockSpec((DIM, TCOLS), lambda i: (0, i))],
        out_specs=pl.BlockSpec((TCOLS, DIM_PAD), lambda i: (i, 0)),
        out_shape=jax.ShapeDtypeStruct((VOCAB, DIM_PAD), jnp.float32),
    )(table_t)


def _sc_body(wids_r, cids_r, wtab_r, ctab_r, out_r,
             idxw_v, idxc_v, wrows0, crows0, wrows1, crows1, outv,
             semw, semc):
    c = lax.axis_index("c")
    s = lax.axis_index("s")
    wid = s * NC + c
    lane15 = lax.iota(jnp.int32, LANES) == (LANES - 1)

    pltpu.sync_copy(wids_r.at[pl.ds(wid * BPW, BPW)], idxw_v)
    pltpu.sync_copy(cids_r.at[pl.ds(wid * BPW * L, BPW * L)], idxc_v)

    def fire(k, wrows_v, crows_v):
        pltpu.async_copy(wtab_r.at[idxw_v.at[pl.ds(k * CB, CB)]],
                         wrows_v, semw)
        for j in range(CTX_LAUNCH):
            pltpu.async_copy(
                ctab_r.at[idxc_v.at[pl.ds(k * OUT_PER_CHUNK
                                          + j * CTX_PER_LAUNCH,
                                          CTX_PER_LAUNCH)]],
                crows_v.at[pl.ds(j * CTX_PER_LAUNCH, CTX_PER_LAUNCH)], semc)

    def drain(k, wrows_v, crows_v):
        pltpu.make_async_copy(wtab_r.at[idxw_v.at[pl.ds(k * CB, CB)]],
                              wrows_v, semw).wait()
        for j in range(CTX_LAUNCH):
            pltpu.make_async_copy(
                ctab_r.at[idxc_v.at[pl.ds(j * CTX_PER_LAUNCH,
                                          CTX_PER_LAUNCH)]],
                crows_v.at[pl.ds(j * CTX_PER_LAUNCH, CTX_PER_LAUNCH)],
                semc).wait()

    def compute(k, wrows_v, crows_v):
        def word_body(b, carry2):
            wv = [wrows_v[b, pl.ds(h * LANES, LANES)] for h in range(4)]
            for l in range(L):
                o = b * L + l
                p01 = (wv[0] * crows_v[o, pl.ds(0, LANES)]
                       + wv[1] * crows_v[o, pl.ds(LANES, LANES)])
                p23 = (wv[2] * crows_v[o, pl.ds(2 * LANES, LANES)]
                       + wv[3] * crows_v[o, pl.ds(3 * LANES, LANES)])
                cum = plsc.cumsum(p01 + p23)
                plsc.store_scatter(
                    outv, [jnp.broadcast_to(o, (LANES,))], cum, mask=lane15)
            return carry2

        lax.fori_loop(0, CB, word_body, 0)
        g = wid * NCHUNK + k
        pltpu.sync_copy(outv, out_r.at[pl.ds(g * OUT_PER_CHUNK, OUT_PER_CHUNK)])

    fire(0, wrows0, crows0)

    def pair_body(jj, carry):
        a = 2 * jj
        drain(a, wrows0, crows0)
        fire(a + 1, wrows1, crows1)
        compute(a, wrows0, crows0)
        drain(a + 1, wrows1, crows1)

        @pl.when(jj < NCHUNK // 2 - 1)
        def _():
            fire(a + 2, wrows0, crows0)

        compute(a + 1, wrows1, crows1)
        return carry

    lax.fori_loop(0, NCHUNK // 2, pair_body, 0)


@jax.jit
def kernel(word_ids, context_ids, word_embed, context_embed):
    wids = word_ids.astype(jnp.int32).reshape(B)
    cids = context_ids.astype(jnp.int32).reshape(B * L)
    wtab = _transpose_pad(word_embed.T)
    ctab = _transpose_pad(context_embed.T)

    mesh = plsc.VectorSubcoreMesh(core_axis_name="c", subcore_axis_name="s")
    out_flat = pl.kernel(
        _sc_body,
        out_type=jax.ShapeDtypeStruct((B * L,), jnp.float32),
        mesh=mesh,
        scratch_types=[
            pltpu.VMEM((BPW,), jnp.int32),
            pltpu.VMEM((BPW * L,), jnp.int32),
            pltpu.VMEM((CB, DIM_PAD), jnp.float32),
            pltpu.VMEM((OUT_PER_CHUNK, DIM_PAD), jnp.float32),
            pltpu.VMEM((CB, DIM_PAD), jnp.float32),
            pltpu.VMEM((OUT_PER_CHUNK, DIM_PAD), jnp.float32),
            pltpu.VMEM((OUT_PER_CHUNK,), jnp.float32),
            pltpu.SemaphoreType.DMA,
            pltpu.SemaphoreType.DMA,
        ],
        compiler_params=pltpu.CompilerParams(
            needs_layout_passes=False, use_tc_tiling_on_sc=True),
    )(wids, cids, wtab, ctab)
    return out_flat.reshape(B, L)


# speedup vs baseline: 3.9084x; 1.0567x over previous
import jax
import jax.numpy as jnp
from jax import lax
from jax.experimental import pallas as pl
from jax.experimental.pallas import tpu as pltpu
from jax.experimental.pallas import tpu_sc as plsc

VOCAB = 1000000
DIM = 64
DIM_PAD = 128
B = 16384
L = 20

NC = 2
NS = 16
LANES = 16
NW = NC * NS

BPW = B // NW
CB = 16
NCHUNK = BPW // CB
OUT_PER_CHUNK = CB * L
CTX_PER_LAUNCH = 64
CTX_LAUNCH = OUT_PER_CHUNK // CTX_PER_LAUNCH

TCOLS = 16384
TGRID = (VOCAB + TCOLS - 1) // TCOLS


def _tp_body(x_ref, o_ref):
    t = jnp.transpose(x_ref[...], (1, 0))
    o_ref[...] = jnp.concatenate(
        [t, jnp.zeros((TCOLS, DIM_PAD - DIM), jnp.float32)], axis=1)


def _transpose_pad(table_t):
    return pl.pallas_call(
        _tp_body,
        grid=(TGRID,),
        in_specs=[pl.BlockSpec((DIM, TCOLS), lambda i: (0, i))],
        out_specs=pl.BlockSpec((TCOLS, DIM_PAD), lambda i: (i, 0)),
        out_shape=jax.ShapeDtypeStruct((VOCAB, DIM_PAD), jnp.float32),
    )(table_t)


def _sc_body(wids_r, cids_r, wtab_r, ctab_r, out_r,
             idxw_v, idxc_v, wrows0, crows0, wrows1, crows1, outv,
             semw, semc):
    c = lax.axis_index("c")
    s = lax.axis_index("s")
    wid = s * NC + c
    lane15 = lax.iota(jnp.int32, LANES) == (LANES - 1)

    pltpu.sync_copy(wids_r.at[pl.ds(wid * BPW, BPW)], idxw_v)
    pltpu.sync_copy(cids_r.at[pl.ds(wid * BPW * L, BPW * L)], idxc_v)

    def fire(k, wrows_v, crows_v):
        pltpu.async_copy(wtab_r.at[idxw_v.at[pl.ds(k * CB, CB)]],
                         wrows_v, semw)
        for j in range(CTX_LAUNCH):
            pltpu.async_copy(
                ctab_r.at[idxc_v.at[pl.ds(k * OUT_PER_CHUNK
                                          + j * CTX_PER_LAUNCH,
                                          CTX_PER_LAUNCH)]],
                crows_v.at[pl.ds(j * CTX_PER_LAUNCH, CTX_PER_LAUNCH)], semc)

    def drain(k, wrows_v, crows_v):
        pltpu.make_async_copy(wtab_r.at[idxw_v.at[pl.ds(k * CB, CB)]],
                              wrows_v, semw).wait()
        for j in range(CTX_LAUNCH):
            pltpu.make_async_copy(
                ctab_r.at[idxc_v.at[pl.ds(j * CTX_PER_LAUNCH,
                                          CTX_PER_LAUNCH)]],
                crows_v.at[pl.ds(j * CTX_PER_LAUNCH, CTX_PER_LAUNCH)],
                semc).wait()

    def compute(k, wrows_v, crows_v):
        def word_body(b, carry2):
            wv = [wrows_v[b, pl.ds(h * LANES, LANES)] for h in range(4)]
            for l in range(L):
                o = b * L + l
                p01 = (wv[0] * crows_v[o, pl.ds(0, LANES)]
                       + wv[1] * crows_v[o, pl.ds(LANES, LANES)])
                p23 = (wv[2] * crows_v[o, pl.ds(2 * LANES, LANES)]
                       + wv[3] * crows_v[o, pl.ds(3 * LANES, LANES)])
                cum = plsc.cumsum(p01 + p23)
                plsc.store_scatter(
                    outv, [jnp.broadcast_to(o, (LANES,))], cum, mask=lane15)
            return carry2

        lax.fori_loop(0, CB, word_body, 0)
        g = wid * NCHUNK + k
        pltpu.sync_copy(outv, out_r.at[pl.ds(g * OUT_PER_CHUNK, OUT_PER_CHUNK)])

    fire(0, wrows0, crows0)

    def pair_body(jj, carry):
        a = 2 * jj
        drain(a, wrows0, crows0)
        fire(a + 1, wrows1, crows1)
        compute(a, wrows0, crows0)
        drain(a + 1, wrows1, crows1)

        @pl.when(jj < NCHUNK // 2 - 1)
        def _():
            fire(a + 2, wrows0, crows0)

        compute(a + 1, wrows1, crows1)
        return carry

    lax.fori_loop(0, NCHUNK // 2, pair_body, 0)


@jax.jit
def kernel(word_ids, context_ids, word_embed, context_embed):
    wids = word_ids.astype(jnp.int32).reshape(B)
    cids = context_ids.astype(jnp.int32).reshape(B * L)
    wtab = _transpose_pad(word_embed.T)
    ctab = _transpose_pad(context_embed.T)

    mesh = plsc.VectorSubcoreMesh(core_axis_name="c", subcore_axis_name="s")
    out_flat = pl.kernel(
        _sc_body,
        out_type=jax.ShapeDtypeStruct((B * L,), jnp.float32),
        mesh=mesh,
        scratch_types=[
            pltpu.VMEM((BPW,), jnp.int32),
            pltpu.VMEM((BPW * L,), jnp.int32),
            pltpu.VMEM((CB, DIM_PAD), jnp.float32),
            pltpu.VMEM((OUT_PER_CHUNK, DIM_PAD), jnp.float32),
            pltpu.VMEM((CB, DIM_PAD), jnp.float32),
            pltpu.VMEM((OUT_PER_CHUNK, DIM_PAD), jnp.float32),
            pltpu.VMEM((OUT_PER_CHUNK,), jnp.float32),
            pltpu.SemaphoreType.DMA,
            pltpu.SemaphoreType.DMA,
        ],
        compiler_params=pltpu.CompilerParams(
            needs_layout_passes=False, use_tc_tiling_on_sc=True),
    )(wids, cids, wtab, ctab)
    return out_flat.reshape(B, L)


# speedup vs baseline: 4.1072x; 1.0509x over previous
import jax
import jax.numpy as jnp
from jax import lax
from jax.experimental import pallas as pl
from jax.experimental.pallas import tpu as pltpu
from jax.experimental.pallas import tpu_sc as plsc

VOCAB = 1000000
DIM = 64
DIM_PAD = 128
B = 16384
L = 20

NC = 2
NS = 16
LANES = 16
NW = NC * NS

BPW = B // NW
CB = 16
NCHUNK = BPW // CB
OUT_PER_CHUNK = CB * L
CTX_PER_LAUNCH = 64
CTX_LAUNCH = OUT_PER_CHUNK // CTX_PER_LAUNCH
FLUSH_EVERY = 128 // CB

TCOLS = 16384
TGRID = (VOCAB + TCOLS - 1) // TCOLS


def _tp_body(x_ref, o_ref):
    t = jnp.transpose(x_ref[...], (1, 0))
    o_ref[...] = jnp.concatenate(
        [t, jnp.zeros((TCOLS, DIM_PAD - DIM), jnp.float32)], axis=1)


def _transpose_pad(table_t):
    return pl.pallas_call(
        _tp_body,
        grid=(TGRID,),
        in_specs=[pl.BlockSpec((DIM, TCOLS), lambda i: (0, i))],
        out_specs=pl.BlockSpec((TCOLS, DIM_PAD), lambda i: (i, 0)),
        out_shape=jax.ShapeDtypeStruct((VOCAB, DIM_PAD), jnp.float32),
    )(table_t)


def _sc_body(wids_r, cids_r, wtab_r, ctab_r, out_r,
             idxw_v, idxc2d_v, idxc_v, wrows0, crows0, wrows1, crows1,
             outv, semw, semc):
    c = lax.axis_index("c")
    s = lax.axis_index("s")
    wid = s * NC + c
    lane15 = lax.iota(jnp.int32, LANES) == (LANES - 1)

    pltpu.sync_copy(wids_r.at[pl.ds(wid * BPW, BPW)], idxw_v)
    pltpu.sync_copy(cids_r.at[pl.ds(0, L), pl.ds(wid * BPW, BPW)], idxc2d_v)
    iota16 = lax.iota(jnp.int32, LANES)

    def reorder_body(i, carry):
        dest = (i * LANES + iota16) * L
        for l in range(L):
            v = idxc2d_v[l, pl.ds(i * LANES, LANES)]
            plsc.store_scatter(idxc_v, [dest + l], v)
        return carry

    lax.fori_loop(0, BPW // LANES, reorder_body, 0)

    def fire(k, wrows_v, crows_v):
        pltpu.async_copy(wtab_r.at[idxw_v.at[pl.ds(k * CB, CB)]],
                         wrows_v, semw)
        for j in range(CTX_LAUNCH):
            pltpu.async_copy(
                ctab_r.at[idxc_v.at[pl.ds(k * OUT_PER_CHUNK
                                          + j * CTX_PER_LAUNCH,
                                          CTX_PER_LAUNCH)]],
                crows_v.at[pl.ds(j * CTX_PER_LAUNCH, CTX_PER_LAUNCH)], semc)

    def drain(k, wrows_v, crows_v):
        pltpu.make_async_copy(wtab_r.at[idxw_v.at[pl.ds(k * CB, CB)]],
                              wrows_v, semw).wait()
        for j in range(CTX_LAUNCH):
            pltpu.make_async_copy(
                ctab_r.at[idxc_v.at[pl.ds(j * CTX_PER_LAUNCH,
                                          CTX_PER_LAUNCH)]],
                crows_v.at[pl.ds(j * CTX_PER_LAUNCH, CTX_PER_LAUNCH)],
                semc).wait()

    def compute(k, wrows_v, crows_v):
        boff = lax.rem(k, FLUSH_EVERY) * CB

        def word_body(b, carry2):
            wv = [wrows_v[b, pl.ds(h * LANES, LANES)] for h in range(4)]
            for l in range(L):
                o = b * L + l
                p01 = (wv[0] * crows_v[o, pl.ds(0, LANES)]
                       + wv[1] * crows_v[o, pl.ds(LANES, LANES)])
                p23 = (wv[2] * crows_v[o, pl.ds(2 * LANES, LANES)]
                       + wv[3] * crows_v[o, pl.ds(3 * LANES, LANES)])
                cum = plsc.cumsum(p01 + p23)
                dcol = jnp.broadcast_to(boff + b, (LANES,))
                drow = jnp.full((LANES,), l, jnp.int32)
                plsc.store_scatter(outv, [drow, dcol], cum, mask=lane15)
            return carry2

        lax.fori_loop(0, CB, word_body, 0)

        @pl.when(lax.rem(k, FLUSH_EVERY) == FLUSH_EVERY - 1)
        def _():
            col = wid * BPW + (k // FLUSH_EVERY) * 128
            pltpu.sync_copy(outv, out_r.at[pl.ds(0, L), pl.ds(col, 128)])

    fire(0, wrows0, crows0)

    def pair_body(jj, carry):
        a = 2 * jj
        drain(a, wrows0, crows0)
        fire(a + 1, wrows1, crows1)
        compute(a, wrows0, crows0)
        drain(a + 1, wrows1, crows1)

        @pl.when(jj < NCHUNK // 2 - 1)
        def _():
            fire(a + 2, wrows0, crows0)

        compute(a + 1, wrows1, crows1)
        return carry

    lax.fori_loop(0, NCHUNK // 2, pair_body, 0)


@jax.jit
def kernel(word_ids, context_ids, word_embed, context_embed):
    wids = word_ids.astype(jnp.int32).reshape(B)
    cids = context_ids.astype(jnp.int32).T
    wtab = _transpose_pad(word_embed.T)
    ctab = _transpose_pad(context_embed.T)

    mesh = plsc.VectorSubcoreMesh(core_axis_name="c", subcore_axis_name="s")
    out_flat = pl.kernel(
        _sc_body,
        out_type=jax.ShapeDtypeStruct((L, B), jnp.float32),
        mesh=mesh,
        scratch_types=[
            pltpu.VMEM((BPW,), jnp.int32),
            pltpu.VMEM((L, BPW), jnp.int32),
            pltpu.VMEM((BPW * L,), jnp.int32),
            pltpu.VMEM((CB, DIM_PAD), jnp.float32),
            pltpu.VMEM((OUT_PER_CHUNK, DIM_PAD), jnp.float32),
            pltpu.VMEM((CB, DIM_PAD), jnp.float32),
            pltpu.VMEM((OUT_PER_CHUNK, DIM_PAD), jnp.float32),
            pltpu.VMEM((L, 128), jnp.float32),
            pltpu.SemaphoreType.DMA,
            pltpu.SemaphoreType.DMA,
        ],
        compiler_params=pltpu.CompilerParams(
            needs_layout_passes=False, use_tc_tiling_on_sc=True),
    )(wids, cids, wtab, ctab)
    return out_flat.T


# speedup vs baseline: 4.1952x; 1.0214x over previous
import jax
import jax.numpy as jnp
from jax import lax
from jax.experimental import pallas as pl
from jax.experimental.pallas import tpu as pltpu
from jax.experimental.pallas import tpu_sc as plsc

VOCAB = 1000000
DIM = 64
DIM_PAD = 128
B = 16384
L = 20

NC = 2
NS = 16
LANES = 16
NW = NC * NS

BPW = B // NW
CB = 16
NCHUNK = BPW // CB
OUT_PER_CHUNK = CB * L
CTX_PER_LAUNCH = 64
CTX_LAUNCH = OUT_PER_CHUNK // CTX_PER_LAUNCH
FLUSH_EVERY = 128 // CB

TCOLS = 32768
TGRID = (VOCAB + TCOLS - 1) // TCOLS


def _tp_body(x_ref, o_ref):
    t = jnp.transpose(x_ref[...], (1, 0))
    o_ref[...] = jnp.concatenate(
        [t, jnp.zeros((TCOLS, DIM_PAD - DIM), jnp.float32)], axis=1)


def _transpose_pad(table_t):
    return pl.pallas_call(
        _tp_body,
        grid=(TGRID,),
        in_specs=[pl.BlockSpec((DIM, TCOLS), lambda i: (0, i))],
        out_specs=pl.BlockSpec((TCOLS, DIM_PAD), lambda i: (i, 0)),
        out_shape=jax.ShapeDtypeStruct((VOCAB, DIM_PAD), jnp.float32),
    )(table_t)


def _sc_body(wids_r, cids_r, wtab_r, ctab_r, out_r,
             idxw_v, idxc2d_v, idxc_v, wrows0, crows0, wrows1, crows1,
             outv, semw, semc):
    c = lax.axis_index("c")
    s = lax.axis_index("s")
    wid = s * NC + c
    lane15 = lax.iota(jnp.int32, LANES) == (LANES - 1)

    pltpu.sync_copy(wids_r.at[pl.ds(wid * BPW, BPW)], idxw_v)
    pltpu.sync_copy(cids_r.at[pl.ds(0, L), pl.ds(wid * BPW, BPW)], idxc2d_v)
    iota16 = lax.iota(jnp.int32, LANES)

    def reorder_body(i, carry):
        dest = (i * LANES + iota16) * L
        for l in range(L):
            v = idxc2d_v[l, pl.ds(i * LANES, LANES)]
            plsc.store_scatter(idxc_v, [dest + l], v)
        return carry

    lax.fori_loop(0, BPW // LANES, reorder_body, 0)

    def fire(k, wrows_v, crows_v):
        pltpu.async_copy(wtab_r.at[idxw_v.at[pl.ds(k * CB, CB)]],
                         wrows_v, semw)
        for j in range(CTX_LAUNCH):
            pltpu.async_copy(
                ctab_r.at[idxc_v.at[pl.ds(k * OUT_PER_CHUNK
                                          + j * CTX_PER_LAUNCH,
                                          CTX_PER_LAUNCH)]],
                crows_v.at[pl.ds(j * CTX_PER_LAUNCH, CTX_PER_LAUNCH)], semc)

    def drain(k, wrows_v, crows_v):
        pltpu.make_async_copy(wtab_r.at[idxw_v.at[pl.ds(k * CB, CB)]],
                              wrows_v, semw).wait()
        for j in range(CTX_LAUNCH):
            pltpu.make_async_copy(
                ctab_r.at[idxc_v.at[pl.ds(j * CTX_PER_LAUNCH,
                                          CTX_PER_LAUNCH)]],
                crows_v.at[pl.ds(j * CTX_PER_LAUNCH, CTX_PER_LAUNCH)],
                semc).wait()

    def compute(k, wrows_v, crows_v):
        boff = lax.rem(k, FLUSH_EVERY) * CB

        def word_body(b, carry2):
            wv = [wrows_v[b, pl.ds(h * LANES, LANES)] for h in range(4)]
            for l in range(L):
                o = b * L + l
                p01 = (wv[0] * crows_v[o, pl.ds(0, LANES)]
                       + wv[1] * crows_v[o, pl.ds(LANES, LANES)])
                p23 = (wv[2] * crows_v[o, pl.ds(2 * LANES, LANES)]
                       + wv[3] * crows_v[o, pl.ds(3 * LANES, LANES)])
                cum = plsc.cumsum(p01 + p23)
                dcol = jnp.broadcast_to(boff + b, (LANES,))
                drow = jnp.full((LANES,), l, jnp.int32)
                plsc.store_scatter(outv, [drow, dcol], cum, mask=lane15)
            return carry2

        lax.fori_loop(0, CB, word_body, 0)

        @pl.when(lax.rem(k, FLUSH_EVERY) == FLUSH_EVERY - 1)
        def _():
            col = wid * BPW + (k // FLUSH_EVERY) * 128
            pltpu.sync_copy(outv, out_r.at[pl.ds(0, L), pl.ds(col, 128)])

    fire(0, wrows0, crows0)

    def pair_body(jj, carry):
        a = 2 * jj
        drain(a, wrows0, crows0)
        fire(a + 1, wrows1, crows1)
        compute(a, wrows0, crows0)
        drain(a + 1, wrows1, crows1)

        @pl.when(jj < NCHUNK // 2 - 1)
        def _():
            fire(a + 2, wrows0, crows0)

        compute(a + 1, wrows1, crows1)
        return carry

    lax.fori_loop(0, NCHUNK // 2, pair_body, 0)


@jax.jit
def kernel(word_ids, context_ids, word_embed, context_embed):
    wids = word_ids.astype(jnp.int32).reshape(B)
    cids = context_ids.astype(jnp.int32).T
    wtab = _transpose_pad(word_embed.T)
    ctab = _transpose_pad(context_embed.T)

    mesh = plsc.VectorSubcoreMesh(core_axis_name="c", subcore_axis_name="s")
    out_flat = pl.kernel(
        _sc_body,
        out_type=jax.ShapeDtypeStruct((L, B), jnp.float32),
        mesh=mesh,
        scratch_types=[
            pltpu.VMEM((BPW,), jnp.int32),
            pltpu.VMEM((L, BPW), jnp.int32),
            pltpu.VMEM((BPW * L,), jnp.int32),
            pltpu.VMEM((CB, DIM_PAD), jnp.float32),
            pltpu.VMEM((OUT_PER_CHUNK, DIM_PAD), jnp.float32),
            pltpu.VMEM((CB, DIM_PAD), jnp.float32),
            pltpu.VMEM((OUT_PER_CHUNK, DIM_PAD), jnp.float32),
            pltpu.VMEM((L, 128), jnp.float32),
            pltpu.SemaphoreType.DMA,
            pltpu.SemaphoreType.DMA,
        ],
        compiler_params=pltpu.CompilerParams(
            needs_layout_passes=False, use_tc_tiling_on_sc=True),
    )(wids, cids, wtab, ctab)
    return out_flat.T
